# baseline (speedup 1.0000x reference)
import jax
import jax.numpy as jnp
from jax import lax
from jax.experimental import pallas as pl
from jax.experimental.pallas import tpu as pltpu
from jax.experimental.pallas import tpu_sc as plsc

K = 3
B, H, W, C = 2, 224, 224, 96
WCHUNK = 112
NCHUNKS = W // WCHUNK
TILES_TOTAL = B * H * NCHUNKS
NWORKERS = 32
PER_WORKER = TILES_TOTAL // NWORKERS
HALO = WCHUNK + K - 1
VALID = WCHUNK + 1


def _decode(t):
    b = t // (H * NCHUNKS)
    rem = t - b * (H * NCHUNKS)
    r = rem // NCHUNKS
    cc = rem - r * NCHUNKS
    return b, r, cc


def _body(images_hbm, zrow_hbm, *rest):
    outs = rest[:K * K]
    buf0, buf1, gsem0, gsem1, ssem0, ssem1 = rest[K * K:]
    cid = lax.axis_index("c")
    sid = lax.axis_index("s")
    wid = sid * 2 + cid

    zv = jnp.zeros((16,), jnp.float32)
    for bf in (buf0, buf1):
        for u in range(K):
            for col in (0, HALO - 1):
                for kk in range(C // 16):
                    bf[u, col, pl.ds(16 * kk, 16)] = zv

    def gather(t, buf, sem):
        b, r, cc = _decode(t)
        c0 = cc * WCHUNK
        s_in = c0 - cc
        d0 = 1 - cc
        handles = []
        for u in range(K):
            dst = buf.at[u, pl.ds(d0, VALID), :]
            if u == 1:
                handles.append(
                    pltpu.async_copy(
                        images_hbm.at[b, r, pl.ds(s_in, VALID), :], dst, sem))
            else:
                row = r - 1 + u
                ok = (row >= 0) if u == 0 else (row < H)
                hs = []

                @pl.when(ok)
                def _(row=row, dst=dst, hs=hs):
                    hs.append(
                        pltpu.async_copy(
                            images_hbm.at[b, row, pl.ds(s_in, VALID), :],
                            dst, sem))

                @pl.when(jnp.logical_not(ok))
                def _(dst=dst):
                    pltpu.async_copy(zrow_hbm, dst, sem)

                handles.append(hs[0])
        return handles

    def stores(t, buf, sem):
        b, r, cc = _decode(t)
        c0 = cc * WCHUNK
        return [
            pltpu.async_copy(
                buf.at[u, pl.ds(v, WCHUNK), :],
                outs[u * K + v].at[b, r, pl.ds(c0, WCHUNK), :],
                sem)
            for u in range(K) for v in range(K)
        ]

    def step(j, carry):
        ta = (2 * j) * NWORKERS + wid
        tb = (2 * j + 1) * NWORKERS + wid
        ha = gather(ta, buf0, gsem0)
        hb = gather(tb, buf1, gsem1)
        for h in ha:
            h.wait()
        hs_a = stores(ta, buf0, ssem0)
        for h in hb:
            h.wait()
        hs_b = stores(tb, buf1, ssem1)
        for h in hs_a:
            h.wait()
        for h in hs_b:
            h.wait()
        return carry

    lax.fori_loop(0, PER_WORKER // 2, step, 0)


@jax.jit
def kernel(images):
    zrow = jnp.zeros((VALID, C), jnp.float32)
    run = pl.kernel(
        _body,
        out_type=tuple(
            jax.ShapeDtypeStruct((B, H, W, C), jnp.float32)
            for _ in range(K * K)),
        mesh=plsc.VectorSubcoreMesh(core_axis_name="c", subcore_axis_name="s"),
        scratch_types=[
            pltpu.VMEM((K, HALO, C), jnp.float32),
            pltpu.VMEM((K, HALO, C), jnp.float32),
            pltpu.SemaphoreType.DMA,
            pltpu.SemaphoreType.DMA,
            pltpu.SemaphoreType.DMA,
            pltpu.SemaphoreType.DMA,
        ],
        compiler_params=pltpu.CompilerParams(use_tc_tiling_on_sc=False),
    )
    return jnp.concatenate(run(images, zrow), axis=3)


# speedup vs baseline: 1.5757x; 1.5757x over previous
import jax
import jax.numpy as jnp
from jax import lax
from jax.experimental import pallas as pl
from jax.experimental.pallas import tpu as pltpu
from jax.experimental.pallas import tpu_sc as plsc

K = 3
B, H, W, C = 2, 224, 224, 96
WCHUNK = 112
NCHUNKS = W // WCHUNK
TILES_TOTAL = B * H * NCHUNKS
NWORKERS = 32
PER_WORKER = TILES_TOTAL // NWORKERS
HALO = WCHUNK + K - 1
VALID = WCHUNK + 1


def _decode(t):
    b = t // (H * NCHUNKS)
    rem = t - b * (H * NCHUNKS)
    r = rem // NCHUNKS
    cc = rem - r * NCHUNKS
    return b, r, cc


def _body(images_hbm, zrow_hbm, out_hbm, buf0, buf1, gsem0, gsem1, ssem0,
          ssem1):
    cid = lax.axis_index("c")
    sid = lax.axis_index("s")
    wid = sid * 2 + cid

    zv = jnp.zeros((16,), jnp.float32)
    for bf in (buf0, buf1):
        for u in range(K):
            for col in (0, HALO - 1):
                for kk in range(C // 16):
                    bf[u, col, pl.ds(16 * kk, 16)] = zv

    def gather(t, buf, sem):
        b, r, cc = _decode(t)
        c0 = cc * WCHUNK
        s_in = c0 - cc
        d0 = 1 - cc
        handles = []
        for u in range(K):
            dst = buf.at[u, pl.ds(d0, VALID), :]
            if u == 1:
                handles.append(
                    pltpu.async_copy(
                        images_hbm.at[b, r, pl.ds(s_in, VALID), :], dst, sem))
            else:
                row = r - 1 + u
                ok = (row >= 0) if u == 0 else (row < H)
                hs = []

                @pl.when(ok)
                def _(row=row, dst=dst, hs=hs):
                    hs.append(
                        pltpu.async_copy(
                            images_hbm.at[b, row, pl.ds(s_in, VALID), :],
                            dst, sem))

                @pl.when(jnp.logical_not(ok))
                def _(dst=dst):
                    pltpu.async_copy(zrow_hbm, dst, sem)

                handles.append(hs[0])
        return handles

    def stores(t, buf, sem):
        b, r, cc = _decode(t)
        c0 = cc * WCHUNK
        return [
            pltpu.async_copy(
                buf.at[u, pl.ds(v, WCHUNK), :],
                out_hbm.at[b, r, pl.ds(c0, WCHUNK), pl.ds((u * K + v) * C, C)],
                sem)
            for u in range(K) for v in range(K)
        ]

    def step(j, carry):
        ta = (2 * j) * NWORKERS + wid
        tb = (2 * j + 1) * NWORKERS + wid
        ha = gather(ta, buf0, gsem0)
        hb = gather(tb, buf1, gsem1)
        for h in ha:
            h.wait()
        hs_a = stores(ta, buf0, ssem0)
        for h in hb:
            h.wait()
        hs_b = stores(tb, buf1, ssem1)
        for h in hs_a:
            h.wait()
        for h in hs_b:
            h.wait()
        return carry

    lax.fori_loop(0, PER_WORKER // 2, step, 0)


@jax.jit
def kernel(images):
    zrow = jnp.zeros((VALID, C), jnp.float32)
    run = pl.kernel(
        _body,
        out_type=jax.ShapeDtypeStruct((B, H, W, K * K * C), jnp.float32),
        mesh=plsc.VectorSubcoreMesh(core_axis_name="c", subcore_axis_name="s"),
        scratch_types=[
            pltpu.VMEM((K, HALO, C), jnp.float32),
            pltpu.VMEM((K, HALO, C), jnp.float32),
            pltpu.SemaphoreType.DMA,
            pltpu.SemaphoreType.DMA,
            pltpu.SemaphoreType.DMA,
            pltpu.SemaphoreType.DMA,
        ],
        compiler_params=pltpu.CompilerParams(use_tc_tiling_on_sc=False),
    )
    return run(images, zrow)
